# baseline (speedup 1.0000x reference)
import functools

import jax
import jax.numpy as jnp
from jax import lax
from jax.experimental import pallas as pl
from jax.experimental.pallas import tpu as pltpu
from jax.experimental.pallas import tpu_sc as plsc

N = 10000
NPAD = 10240
E = 320000
D = 128
DH = D // 2
NC = 2
NS = 16
NW = NC * NS
EW = E // NW
ET = E // NS
C = 100
NCH = ET // C
RPT = NPAD // NS

_mesh = plsc.VectorSubcoreMesh(core_axis_name="c", subcore_axis_name="s")


@functools.partial(
    pl.kernel,
    out_type=jax.ShapeDtypeStruct((NW, NPAD), jnp.float32),
    mesh=_mesh,
    scratch_types=[
        pltpu.VMEM((EW,), jnp.int32),
        pltpu.VMEM((NPAD,), jnp.float32),
    ],
    compiler_params=pltpu.CompilerParams(needs_layout_passes=False),
)
def _sc_degree(dst_hbm, out_hbm, dst_v, hist_v):
    c = lax.axis_index("c")
    s = lax.axis_index("s")
    wid = s * NC + c
    pltpu.sync_copy(dst_hbm.at[wid], dst_v)
    zeros = jnp.zeros((16,), jnp.float32)

    def zbody(i, carry):
        hist_v[pl.ds(i * 16, 16)] = zeros
        return carry

    lax.fori_loop(0, NPAD // 16, zbody, 0)
    ones = jnp.ones((16,), jnp.float32)

    def hbody(i, carry):
        idx = dst_v[pl.ds(i * 16, 16)]
        plsc.addupdate_scatter(hist_v, [idx], ones)
        return carry

    lax.fori_loop(0, EW // 16, hbody, 0)
    pltpu.sync_copy(hist_v, out_hbm.at[wid])


@functools.partial(
    pl.kernel,
    out_type=jax.ShapeDtypeStruct((NC, NPAD, DH), jnp.float32),
    mesh=_mesh,
    scratch_types=[
        pltpu.VMEM((NCH, C), jnp.int32),
        pltpu.VMEM((NCH, C), jnp.int32),
        pltpu.VMEM((C, DH), jnp.float32),
        pltpu.VMEM((C, DH), jnp.float32),
        pltpu.VMEM_SHARED((NPAD, DH), jnp.float32),
        pltpu.SemaphoreType.DMA,
    ],
    compiler_params=pltpu.CompilerParams(
        needs_layout_passes=False, use_tc_tiling_on_sc=False),
)
def _sc_aggregate(table_hbm, sd_hbm, zeros_hbm, out_hbm,
                  src_v, dst_v, rows_a, rows_b, acc_sh, sem):
    c = lax.axis_index("c")
    s = lax.axis_index("s")
    r0 = s * RPT
    pltpu.sync_copy(zeros_hbm.at[pl.ds(r0, RPT)], acc_sh.at[pl.ds(r0, RPT)])
    pltpu.sync_copy(sd_hbm.at[c, s, 0], src_v)
    pltpu.sync_copy(sd_hbm.at[c, s, 1], dst_v)
    plsc.subcore_barrier()

    pltpu.async_copy(table_hbm.at[src_v.at[0]], rows_a, sem)

    def body(g, carry):
        j0 = g * 2
        pltpu.make_async_copy(
            table_hbm.at[src_v.at[j0]], rows_a, sem).wait()
        pltpu.async_copy(table_hbm.at[src_v.at[j0 + 1]], rows_b, sem)
        pltpu.sync_copy(rows_a, acc_sh.at[dst_v.at[j0]], add=True)
        pltpu.make_async_copy(
            table_hbm.at[src_v.at[j0 + 1]], rows_b, sem).wait()

        @pl.when(j0 + 2 < NCH)
        def _():
            pltpu.async_copy(table_hbm.at[src_v.at[j0 + 2]], rows_a, sem)

        pltpu.sync_copy(rows_b, acc_sh.at[dst_v.at[j0 + 1]], add=True)
        return carry

    lax.fori_loop(0, NCH // 2, body, 0)
    plsc.subcore_barrier()
    pltpu.sync_copy(acc_sh.at[pl.ds(r0, RPT)], out_hbm.at[c, pl.ds(r0, RPT)])


def _tc_prep(degp):
    def body(degp_ref, dinv_ref):
        deg = jnp.sum(degp_ref[...], axis=0) + 1.0
        dinv_ref[...] = lax.rsqrt(deg)

    return pl.pallas_call(
        body,
        out_shape=jax.ShapeDtypeStruct((NPAD // D, D), jnp.float32),
    )(degp)


BR = 1024
GR = NPAD // BR

_row_spec = pl.BlockSpec((BR, D), lambda i: (i, 0))
_dv_spec = pl.BlockSpec((BR, 1), lambda i: (i, 0))
_w_spec = pl.BlockSpec((D, D), lambda i: (0, 0))
_b_spec = pl.BlockSpec((1, D), lambda i: (0, 0))
_sp_spec = pl.BlockSpec((NC, BR, DH), lambda i: (0, i, 0))
_row_ty = jax.ShapeDtypeStruct((NPAD, D), jnp.float32)
_sp_ty = jax.ShapeDtypeStruct((NC, NPAD, DH), jnp.float32)


def _split(v):
    return v[:, :DH], v[:, DH:]


def _tc_mm1(x, dinv_col, Wg1, Wm1, bm1):
    def body(x_ref, dv_ref, wg_ref, wm_ref, bm_ref, t1_ref, zm1_ref):
        xb = x_ref[...]
        t1 = jnp.dot(xb, wg_ref[...],
                     preferred_element_type=jnp.float32) * dv_ref[...]
        lo, hi = _split(t1)
        t1_ref[0] = lo
        t1_ref[1] = hi
        zm1_ref[...] = jnp.maximum(
            jnp.dot(xb, wm_ref[...], preferred_element_type=jnp.float32)
            + bm_ref[...], 0.0)

    return pl.pallas_call(
        body,
        grid=(GR,),
        in_specs=[_row_spec, _dv_spec, _w_spec, _w_spec, _b_spec],
        out_specs=[_sp_spec, _row_spec],
        out_shape=[_sp_ty, _row_ty],
    )(x, dinv_col, Wg1, Wm1, bm1)


def _tc_mm2(P, t1, dinv_col, bg1, Wg2):
    def body(p_ref, t1_ref, dv_ref, bg_ref, w_ref, t2_ref):
        agg = jnp.concatenate(
            [p_ref[0] + t1_ref[0], p_ref[1] + t1_ref[1]], axis=-1)
        zg = jnp.maximum(agg * dv_ref[...] + bg_ref[...], 0.0)
        t2 = jnp.dot(zg, w_ref[...],
                     preferred_element_type=jnp.float32) * dv_ref[...]
        lo, hi = _split(t2)
        t2_ref[0] = lo
        t2_ref[1] = hi

    return pl.pallas_call(
        body,
        grid=(GR,),
        in_specs=[_sp_spec, _sp_spec, _dv_spec, _b_spec, _w_spec],
        out_specs=_sp_spec,
        out_shape=_sp_ty,
    )(P, t1, dinv_col, bg1, Wg2)


def _tc_mm3(Q, t2, dinv_col, bg2, zm1, Wm2, bm2):
    def body(q_ref, t2_ref, dv_ref, bg_ref, zm1_ref, wm_ref, bm_ref, o_ref):
        agg = jnp.concatenate(
            [q_ref[0] + t2_ref[0], q_ref[1] + t2_ref[1]], axis=-1)
        zg2 = agg * dv_ref[...] + bg_ref[...]
        zm2 = jnp.dot(zm1_ref[...], wm_ref[...],
                      preferred_element_type=jnp.float32) + bm_ref[...]
        o_ref[...] = 0.5 * zg2 + 0.5 * zm2

    return pl.pallas_call(
        body,
        grid=(GR,),
        in_specs=[_sp_spec, _sp_spec, _dv_spec, _b_spec, _row_spec,
                  _w_spec, _b_spec],
        out_specs=_row_spec,
        out_shape=_row_ty,
    )(Q, t2, dinv_col, bg2, zm1, Wm2, bm2)


def kernel(x, edge_index, Wg1, bg1, Wg2, bg2, Wm1, bm1, Wm2, bm2):
    src_r = edge_index[0].reshape(NS, NCH, C)
    dst_r = edge_index[1].reshape(NS, NCH, C)
    sd = jnp.stack([
        jnp.stack([src_r, dst_r], axis=1),
        jnp.stack([src_r + NPAD, dst_r], axis=1),
    ], axis=0)
    dstw = edge_index[1].reshape(NW, EW)
    xpad = jnp.pad(x, ((0, NPAD - N), (0, 0)))
    zeros = jnp.zeros((NPAD, DH), jnp.float32)

    degp = _sc_degree(dstw)
    dinv = _tc_prep(degp.reshape(NW, NPAD // D, D))
    dinv_col = dinv.reshape(NPAD, 1)

    t1, zm1 = _tc_mm1(xpad, dinv_col, Wg1, Wm1, bm1.reshape(1, D))
    P = _sc_aggregate(t1.reshape(NC * NPAD, DH), sd, zeros)
    t2 = _tc_mm2(P, t1, dinv_col, bg1.reshape(1, D), Wg2)
    Q = _sc_aggregate(t2.reshape(NC * NPAD, DH), sd, zeros)
    out = _tc_mm3(Q, t2, dinv_col, bg2.reshape(1, D), zm1,
                  Wm2, bm2.reshape(1, D))
    return out[:N]


# speedup vs baseline: 1.6084x; 1.6084x over previous
import functools

import jax
import jax.numpy as jnp
from jax import lax
from jax.experimental import pallas as pl
from jax.experimental.pallas import tpu as pltpu
from jax.experimental.pallas import tpu_sc as plsc

N = 10000
NPAD = 10240
E = 320000
D = 128
DH = D // 2
NC = 2
NS = 16
NW = NC * NS
EW = E // NW
ET = E // NS
C = 100
NCH = ET // C
RPT = NPAD // NS

_mesh = plsc.VectorSubcoreMesh(core_axis_name="c", subcore_axis_name="s")


@functools.partial(
    pl.kernel,
    out_type=jax.ShapeDtypeStruct((NW, NPAD), jnp.float32),
    mesh=_mesh,
    scratch_types=[
        pltpu.VMEM((EW,), jnp.int32),
        pltpu.VMEM((NPAD,), jnp.float32),
    ],
    compiler_params=pltpu.CompilerParams(needs_layout_passes=False),
)
def _sc_degree(dst_hbm, out_hbm, dst_v, hist_v):
    c = lax.axis_index("c")
    s = lax.axis_index("s")
    wid = s * NC + c
    pltpu.sync_copy(dst_hbm.at[wid], dst_v)
    zeros = jnp.zeros((16,), jnp.float32)

    def zbody(i, carry):
        hist_v[pl.ds(i * 16, 16)] = zeros
        return carry

    lax.fori_loop(0, NPAD // 16, zbody, 0)
    ones = jnp.ones((16,), jnp.float32)

    def hbody(i, carry):
        idx = dst_v[pl.ds(i * 16, 16)]
        plsc.addupdate_scatter(hist_v, [idx], ones)
        return carry

    lax.fori_loop(0, EW // 16, hbody, 0)
    pltpu.sync_copy(hist_v, out_hbm.at[wid])


@functools.partial(
    pl.kernel,
    out_type=jax.ShapeDtypeStruct((NC, NPAD, DH), jnp.float32),
    mesh=_mesh,
    scratch_types=[
        pltpu.VMEM((NCH, C), jnp.int32),
        pltpu.VMEM((NCH, C), jnp.int32),
        pltpu.VMEM((C, DH), jnp.float32),
        pltpu.VMEM((C, DH), jnp.float32),
        pltpu.VMEM((C, DH), jnp.float32),
        pltpu.VMEM((C, DH), jnp.float32),
        pltpu.VMEM_SHARED((NPAD, DH), jnp.float32),
        pltpu.SemaphoreType.DMA,
    ],
    compiler_params=pltpu.CompilerParams(
        needs_layout_passes=False, use_tc_tiling_on_sc=False),
)
def _sc_aggregate(table_hbm, sd_hbm, zeros_hbm, out_hbm,
                  src_v, dst_v, rows_a, rows_b, rows_c, rows_d, acc_sh, sem):
    c = lax.axis_index("c")
    s = lax.axis_index("s")
    r0 = s * RPT
    pltpu.sync_copy(zeros_hbm.at[pl.ds(r0, RPT)], acc_sh.at[pl.ds(r0, RPT)])
    pltpu.sync_copy(sd_hbm.at[c, s, 0], src_v)
    pltpu.sync_copy(sd_hbm.at[c, s, 1], dst_v)
    plsc.subcore_barrier()

    bufs = (rows_a, rows_b, rows_c, rows_d)
    for k in range(3):
        pltpu.async_copy(table_hbm.at[src_v.at[k]], bufs[k], sem)

    def body(g, carry):
        j0 = g * 4
        for k in range(4):
            j = j0 + k
            buf = bufs[k]
            pltpu.make_async_copy(
                table_hbm.at[src_v.at[j]], buf, sem).wait()

            @pl.when(j + 3 < NCH)
            def _():
                pltpu.async_copy(
                    table_hbm.at[src_v.at[j + 3]], bufs[(k + 3) % 4], sem)

            pltpu.sync_copy(buf, acc_sh.at[dst_v.at[j]], add=True)
        return carry

    lax.fori_loop(0, NCH // 4, body, 0)
    plsc.subcore_barrier()
    pltpu.sync_copy(acc_sh.at[pl.ds(r0, RPT)], out_hbm.at[c, pl.ds(r0, RPT)])


def _tc_prep(degp):
    def body(degp_ref, dinv_ref):
        deg = jnp.sum(degp_ref[...], axis=0) + 1.0
        dinv_ref[...] = lax.rsqrt(deg)

    return pl.pallas_call(
        body,
        out_shape=jax.ShapeDtypeStruct((NPAD // D, D), jnp.float32),
    )(degp)


BR = 1024
GR = NPAD // BR

_row_spec = pl.BlockSpec((BR, D), lambda i: (i, 0))
_dv_spec = pl.BlockSpec((BR, 1), lambda i: (i, 0))
_w_spec = pl.BlockSpec((D, D), lambda i: (0, 0))
_b_spec = pl.BlockSpec((1, D), lambda i: (0, 0))
_sp_spec = pl.BlockSpec((NC, BR, DH), lambda i: (0, i, 0))
_row_ty = jax.ShapeDtypeStruct((NPAD, D), jnp.float32)
_sp_ty = jax.ShapeDtypeStruct((NC, NPAD, DH), jnp.float32)


def _split(v):
    return v[:, :DH], v[:, DH:]


def _tc_mm1(x, dinv_col, Wg1, Wm1, bm1):
    def body(x_ref, dv_ref, wg_ref, wm_ref, bm_ref, t1_ref, zm1_ref):
        xb = x_ref[...]
        t1 = jnp.dot(xb, wg_ref[...],
                     preferred_element_type=jnp.float32) * dv_ref[...]
        lo, hi = _split(t1)
        t1_ref[0] = lo
        t1_ref[1] = hi
        zm1_ref[...] = jnp.maximum(
            jnp.dot(xb, wm_ref[...], preferred_element_type=jnp.float32)
            + bm_ref[...], 0.0)

    return pl.pallas_call(
        body,
        grid=(GR,),
        in_specs=[_row_spec, _dv_spec, _w_spec, _w_spec, _b_spec],
        out_specs=[_sp_spec, _row_spec],
        out_shape=[_sp_ty, _row_ty],
    )(x, dinv_col, Wg1, Wm1, bm1)


def _tc_mm2(P, t1, dinv_col, bg1, Wg2):
    def body(p_ref, t1_ref, dv_ref, bg_ref, w_ref, t2_ref):
        agg = jnp.concatenate(
            [p_ref[0] + t1_ref[0], p_ref[1] + t1_ref[1]], axis=-1)
        zg = jnp.maximum(agg * dv_ref[...] + bg_ref[...], 0.0)
        t2 = jnp.dot(zg, w_ref[...],
                     preferred_element_type=jnp.float32) * dv_ref[...]
        lo, hi = _split(t2)
        t2_ref[0] = lo
        t2_ref[1] = hi

    return pl.pallas_call(
        body,
        grid=(GR,),
        in_specs=[_sp_spec, _sp_spec, _dv_spec, _b_spec, _w_spec],
        out_specs=_sp_spec,
        out_shape=_sp_ty,
    )(P, t1, dinv_col, bg1, Wg2)


def _tc_mm3(Q, t2, dinv_col, bg2, zm1, Wm2, bm2):
    def body(q_ref, t2_ref, dv_ref, bg_ref, zm1_ref, wm_ref, bm_ref, o_ref):
        agg = jnp.concatenate(
            [q_ref[0] + t2_ref[0], q_ref[1] + t2_ref[1]], axis=-1)
        zg2 = agg * dv_ref[...] + bg_ref[...]
        zm2 = jnp.dot(zm1_ref[...], wm_ref[...],
                      preferred_element_type=jnp.float32) + bm_ref[...]
        o_ref[...] = 0.5 * zg2 + 0.5 * zm2

    return pl.pallas_call(
        body,
        grid=(GR,),
        in_specs=[_sp_spec, _sp_spec, _dv_spec, _b_spec, _row_spec,
                  _w_spec, _b_spec],
        out_specs=_row_spec,
        out_shape=_row_ty,
    )(Q, t2, dinv_col, bg2, zm1, Wm2, bm2)


def kernel(x, edge_index, Wg1, bg1, Wg2, bg2, Wm1, bm1, Wm2, bm2):
    src_r = edge_index[0].reshape(NS, NCH, C)
    dst_r = edge_index[1].reshape(NS, NCH, C)
    sd = jnp.stack([
        jnp.stack([src_r, dst_r], axis=1),
        jnp.stack([src_r + NPAD, dst_r], axis=1),
    ], axis=0)
    dstw = edge_index[1].reshape(NW, EW)
    xpad = jnp.pad(x, ((0, NPAD - N), (0, 0)))
    zeros = jnp.zeros((NPAD, DH), jnp.float32)

    degp = _sc_degree(dstw)
    dinv = _tc_prep(degp.reshape(NW, NPAD // D, D))
    dinv_col = dinv.reshape(NPAD, 1)

    t1, zm1 = _tc_mm1(xpad, dinv_col, Wg1, Wm1, bm1.reshape(1, D))
    P = _sc_aggregate(t1.reshape(NC * NPAD, DH), sd, zeros)
    t2 = _tc_mm2(P, t1, dinv_col, bg1.reshape(1, D), Wg2)
    Q = _sc_aggregate(t2.reshape(NC * NPAD, DH), sd, zeros)
    out = _tc_mm3(Q, t2, dinv_col, bg2.reshape(1, D), zm1,
                  Wm2, bm2.reshape(1, D))
    return out[:N]


# speedup vs baseline: 2.1516x; 1.3377x over previous
import functools

import jax
import jax.numpy as jnp
from jax import lax
from jax.experimental import pallas as pl
from jax.experimental.pallas import tpu as pltpu
from jax.experimental.pallas import tpu_sc as plsc

N = 10000
NPAD = 10240
E = 320000
D = 128
DH = D // 2
NC = 2
NS = 16
NW = NC * NS
EW = E // NW
ET = E // NS
C = 125
NCH = ET // C
RPT = NPAD // NS

_mesh = plsc.VectorSubcoreMesh(core_axis_name="c", subcore_axis_name="s")


@functools.partial(
    pl.kernel,
    out_type=jax.ShapeDtypeStruct((NW, NPAD), jnp.float32),
    mesh=_mesh,
    scratch_types=[
        pltpu.VMEM((EW,), jnp.int32),
        pltpu.VMEM((NPAD,), jnp.float32),
    ],
    compiler_params=pltpu.CompilerParams(needs_layout_passes=False),
)
def _sc_degree(dst_hbm, out_hbm, dst_v, hist_v):
    c = lax.axis_index("c")
    s = lax.axis_index("s")
    wid = s * NC + c
    pltpu.sync_copy(dst_hbm.at[wid], dst_v)
    zeros = jnp.zeros((16,), jnp.float32)

    def zbody(i, carry):
        hist_v[pl.ds(i * 16, 16)] = zeros
        return carry

    lax.fori_loop(0, NPAD // 16, zbody, 0)
    ones = jnp.ones((16,), jnp.float32)

    def hbody(i, carry):
        idx = dst_v[pl.ds(i * 16, 16)]
        plsc.addupdate_scatter(hist_v, [idx], ones)
        return carry

    lax.fori_loop(0, EW // 16, hbody, 0)
    pltpu.sync_copy(hist_v, out_hbm.at[wid])


@functools.partial(
    pl.kernel,
    out_type=jax.ShapeDtypeStruct((NC, NPAD, DH), jnp.bfloat16),
    mesh=_mesh,
    scratch_types=[
        pltpu.VMEM((NCH, C), jnp.int32),
        pltpu.VMEM((NCH, C), jnp.int32),
        pltpu.VMEM((C, DH), jnp.bfloat16),
        pltpu.VMEM((C, DH), jnp.bfloat16),
        pltpu.VMEM((C, DH), jnp.bfloat16),
        pltpu.VMEM((C, DH), jnp.bfloat16),
        pltpu.VMEM((C, DH), jnp.bfloat16),
        pltpu.VMEM((C, DH), jnp.bfloat16),
        pltpu.VMEM((C, DH), jnp.bfloat16),
        pltpu.VMEM((C, DH), jnp.bfloat16),
        pltpu.VMEM_SHARED((NPAD, DH), jnp.bfloat16),
        pltpu.SemaphoreType.DMA,
    ],
    compiler_params=pltpu.CompilerParams(
        needs_layout_passes=False, use_tc_tiling_on_sc=False),
)
def _sc_aggregate(table_hbm, sd_hbm, zeros_hbm, out_hbm,
                  src_v, dst_v, rows_a, rows_b, rows_c, rows_d,
                  rows_e, rows_f, rows_g, rows_h, acc_sh, sem):
    c = lax.axis_index("c")
    s = lax.axis_index("s")
    r0 = s * RPT
    pltpu.sync_copy(zeros_hbm.at[pl.ds(r0, RPT)], acc_sh.at[pl.ds(r0, RPT)])
    pltpu.sync_copy(sd_hbm.at[c, s, 0], src_v)
    pltpu.sync_copy(sd_hbm.at[c, s, 1], dst_v)
    plsc.subcore_barrier()

    NB = 8
    bufs = (rows_a, rows_b, rows_c, rows_d, rows_e, rows_f, rows_g, rows_h)
    for k in range(NB - 1):
        pltpu.async_copy(table_hbm.at[src_v.at[k]], bufs[k], sem)

    def body(g, carry):
        j0 = g * NB
        for k in range(NB):
            j = j0 + k
            buf = bufs[k]
            pltpu.make_async_copy(
                table_hbm.at[src_v.at[j]], buf, sem).wait()

            @pl.when(j + NB - 1 < NCH)
            def _():
                pltpu.async_copy(
                    table_hbm.at[src_v.at[j + NB - 1]],
                    bufs[(k + NB - 1) % NB], sem)

            pltpu.sync_copy(buf, acc_sh.at[dst_v.at[j]], add=True)
        return carry

    lax.fori_loop(0, NCH // NB, body, 0)
    plsc.subcore_barrier()
    pltpu.sync_copy(acc_sh.at[pl.ds(r0, RPT)], out_hbm.at[c, pl.ds(r0, RPT)])


def _tc_prep(degp):
    def body(degp_ref, dinv_ref):
        deg = jnp.sum(degp_ref[...], axis=0) + 1.0
        dinv_ref[...] = lax.rsqrt(deg)

    return pl.pallas_call(
        body,
        out_shape=jax.ShapeDtypeStruct((NPAD // D, D), jnp.float32),
    )(degp)


BR = 1024
GR = NPAD // BR

_row_spec = pl.BlockSpec((BR, D), lambda i: (i, 0))
_dv_spec = pl.BlockSpec((BR, 1), lambda i: (i, 0))
_w_spec = pl.BlockSpec((D, D), lambda i: (0, 0))
_b_spec = pl.BlockSpec((1, D), lambda i: (0, 0))
_sp_spec = pl.BlockSpec((NC, BR, DH), lambda i: (0, i, 0))
_row_ty = jax.ShapeDtypeStruct((NPAD, D), jnp.float32)
_sp_ty = jax.ShapeDtypeStruct((NC, NPAD, DH), jnp.bfloat16)


def _split(v):
    return v[:, :DH], v[:, DH:]


def _tc_mm1(x, dinv_col, Wg1, Wm1, bm1):
    def body(x_ref, dv_ref, wg_ref, wm_ref, bm_ref, t1_ref, zm1_ref):
        xb = x_ref[...]
        t1 = jnp.dot(xb, wg_ref[...],
                     preferred_element_type=jnp.float32) * dv_ref[...]
        lo, hi = _split(t1.astype(jnp.bfloat16))
        t1_ref[0] = lo
        t1_ref[1] = hi
        zm1_ref[...] = jnp.maximum(
            jnp.dot(xb, wm_ref[...], preferred_element_type=jnp.float32)
            + bm_ref[...], 0.0)

    return pl.pallas_call(
        body,
        grid=(GR,),
        in_specs=[_row_spec, _dv_spec, _w_spec, _w_spec, _b_spec],
        out_specs=[_sp_spec, _row_spec],
        out_shape=[_sp_ty, _row_ty],
    )(x, dinv_col, Wg1, Wm1, bm1)


def _tc_mm2(P, t1, dinv_col, bg1, Wg2):
    def body(p_ref, t1_ref, dv_ref, bg_ref, w_ref, t2_ref):
        agg = jnp.concatenate(
            [p_ref[0] + t1_ref[0], p_ref[1] + t1_ref[1]],
            axis=-1).astype(jnp.float32)
        zg = jnp.maximum(agg * dv_ref[...] + bg_ref[...], 0.0)
        t2 = jnp.dot(zg, w_ref[...],
                     preferred_element_type=jnp.float32) * dv_ref[...]
        lo, hi = _split(t2.astype(jnp.bfloat16))
        t2_ref[0] = lo
        t2_ref[1] = hi

    return pl.pallas_call(
        body,
        grid=(GR,),
        in_specs=[_sp_spec, _sp_spec, _dv_spec, _b_spec, _w_spec],
        out_specs=_sp_spec,
        out_shape=_sp_ty,
    )(P, t1, dinv_col, bg1, Wg2)


def _tc_mm3(Q, t2, dinv_col, bg2, zm1, Wm2, bm2):
    def body(q_ref, t2_ref, dv_ref, bg_ref, zm1_ref, wm_ref, bm_ref, o_ref):
        agg = jnp.concatenate(
            [q_ref[0] + t2_ref[0], q_ref[1] + t2_ref[1]],
            axis=-1).astype(jnp.float32)
        zg2 = agg * dv_ref[...] + bg_ref[...]
        zm2 = jnp.dot(zm1_ref[...], wm_ref[...],
                      preferred_element_type=jnp.float32) + bm_ref[...]
        o_ref[...] = 0.5 * zg2 + 0.5 * zm2

    return pl.pallas_call(
        body,
        grid=(GR,),
        in_specs=[_sp_spec, _sp_spec, _dv_spec, _b_spec, _row_spec,
                  _w_spec, _b_spec],
        out_specs=_row_spec,
        out_shape=_row_ty,
    )(Q, t2, dinv_col, bg2, zm1, Wm2, bm2)


def kernel(x, edge_index, Wg1, bg1, Wg2, bg2, Wm1, bm1, Wm2, bm2):
    src_r = edge_index[0].reshape(NS, NCH, C)
    dst_r = edge_index[1].reshape(NS, NCH, C)
    sd = jnp.stack([
        jnp.stack([src_r, dst_r], axis=1),
        jnp.stack([src_r + NPAD, dst_r], axis=1),
    ], axis=0)
    dstw = edge_index[1].reshape(NW, EW)
    xpad = jnp.pad(x, ((0, NPAD - N), (0, 0)))
    zeros = jnp.zeros((NPAD, DH), jnp.bfloat16)

    degp = _sc_degree(dstw)
    dinv = _tc_prep(degp.reshape(NW, NPAD // D, D))
    dinv_col = dinv.reshape(NPAD, 1)

    t1, zm1 = _tc_mm1(xpad, dinv_col, Wg1, Wm1, bm1.reshape(1, D))
    P = _sc_aggregate(t1.reshape(NC * NPAD, DH), sd, zeros)
    t2 = _tc_mm2(P, t1, dinv_col, bg1.reshape(1, D), Wg2)
    Q = _sc_aggregate(t2.reshape(NC * NPAD, DH), sd, zeros)
    out = _tc_mm3(Q, t2, dinv_col, bg2.reshape(1, D), zm1,
                  Wm2, bm2.reshape(1, D))
    return out[:N]
